# baseline (speedup 1.0000x reference)
import functools

import jax
import jax.numpy as jnp
from jax import lax
from jax.experimental import pallas as pl
from jax.experimental.pallas import tpu as pltpu
from jax.experimental.pallas import tpu_sc as plsc

N = 10000
IN = 128
OUT = 32
S = 4
P = 22
K = 16
TW = 128
AW = 64
BQ = 80
BIG = 1e30
NPAD = 10240
NW = 32
BPW = NPAD // NW
SUB = 40
NSUB = BPW // SUB
_INTERPRET = False


def _bdot(a, b, dn):
    return jax.lax.dot_general(
        a.astype(jnp.bfloat16), b.astype(jnp.bfloat16), dn,
        preferred_element_type=jnp.float32)


def _prep_body(x_ref, ws_ref, bs_ref, wh_ref, bh_ref, w1_ref,
               s_ref, t_ref, xo_ref):
    x = x_ref[...]
    dn = (((1,), (1,)), ((), ()))
    s = _bdot(x, ws_ref[...], dn) + bs_ref[...]
    s_ref[...] = s
    h = _bdot(x, wh_ref[...], dn) + bh_ref[...]
    t_ref[:, 0:P] = h
    t_ref[:, P:P + S] = s
    t_ref[:, P + S:32] = jnp.zeros((x.shape[0], 32 - P - S), jnp.float32)
    t_ref[:, 32:TW] = jnp.zeros((x.shape[0], TW - 32), jnp.float32)
    xo_ref[...] = _bdot(x, w1_ref[...], dn)


def _knn_body(sq_ref, st_ref, bq_ref, bk_ref, nbr_ref):
    sq = sq_ref[...]
    st = st_ref[...]
    qn = jnp.sum(sq * sq, axis=1, keepdims=True)
    kn = jnp.sum(st * st, axis=0, keepdims=True)
    cross = _bdot(sq, st, (((1,), (0,)), ((), ())))
    d2 = qn + kn - 2.0 * cross
    mask = bq_ref[...] != bk_ref[...]
    d = jnp.where(mask, BIG, d2)
    iota = jax.lax.broadcasted_iota(jnp.int32, (BQ, N), 1)
    for k in range(K):
        idx = jnp.argmin(d, axis=1, keepdims=True)
        nbr_ref[:, k:k + 1] = idx.astype(jnp.int32)
        d = jnp.where(iota == idx, BIG, d)


def _take16(v, idx):
    return lax.gather(
        v, idx.reshape(16, 1),
        lax.GatherDimensionNumbers(offset_dims=(), collapsed_slice_dims=(0,),
                                   start_index_map=(0,)),
        (1,), mode=lax.GatherScatterMode.PROMISE_IN_BOUNDS)


def _sc_agg_body(t_hbm, nbr_hbm, out_hbm, idx_v, rows_v, q_rows, out_v, sem):
    wid = lax.axis_index("s") * 2 + lax.axis_index("c")
    base = wid * BPW

    def sub_chunk(j, carry):
        pltpu.sync_copy(nbr_hbm.at[pl.ds((base + j * SUB) * K, SUB * K)], idx_v)
        pltpu.async_copy(t_hbm.at[idx_v], rows_v, sem).wait()
        pltpu.sync_copy(t_hbm.at[pl.ds(base + j * SUB, SUB)], q_rows)

        lanes = lax.iota(jnp.int32, 16)
        rot1 = (lanes + 1) & 15
        rot2 = (lanes + 2) & 15
        rot3 = (lanes + 3) & 15

        def node(i, carry2):
            gi = j * SUB + i
            qv1 = q_rows[i, 16:32]
            acc0 = jnp.zeros((16,), jnp.float32)
            acc1 = jnp.zeros((16,), jnp.float32)
            mx0 = jnp.full((16,), -jnp.inf, jnp.float32)
            mx1 = jnp.full((16,), -jnp.inf, jnp.float32)
            for k in range(K):
                h0 = rows_v[i * K + k, 0:16]
                h1 = rows_v[i * K + k, 16:32]
                diff = h1 - qv1
                dsq = diff * diff
                d2v = (dsq + _take16(dsq, rot1)
                       + _take16(dsq, rot2) + _take16(dsq, rot3))
                w = jnp.exp(-10.0 * d2v)
                wk = w[P - 16]
                m0 = wk * h0
                m1 = wk * h1
                acc0 = acc0 + m0
                acc1 = acc1 + m1
                mx0 = jnp.maximum(mx0, m0)
                mx1 = jnp.maximum(mx1, m1)
            out_v[gi, 0:16] = acc0 * (1.0 / K)
            out_v[gi, 16:32] = acc1 * (1.0 / K)
            out_v[gi, 32:48] = mx0
            out_v[gi, 48:64] = mx1
            return carry2

        lax.fori_loop(0, SUB, node, 0)
        return carry

    lax.fori_loop(0, NSUB, sub_chunk, 0)
    pltpu.sync_copy(out_v, out_hbm.at[pl.ds(base, BPW)])


_sc_agg = functools.partial(
    pl.kernel,
    out_type=jax.ShapeDtypeStruct((NPAD, AW), jnp.float32),
    mesh=plsc.VectorSubcoreMesh(core_axis_name="c", subcore_axis_name="s"),
    scratch_types=[
        pltpu.VMEM((SUB * K,), jnp.int32),
        pltpu.VMEM((SUB * K, TW), jnp.float32),
        pltpu.VMEM((SUB, TW), jnp.float32),
        pltpu.VMEM((BPW, AW), jnp.float32),
        pltpu.SemaphoreType.DMA,
    ],
)(_sc_agg_body)


def _head_body(xo_ref, agg_ref, w2_ref, b2_ref, g1_ref, bb1_ref,
               fw1_ref, fb1_ref, g2_ref, bb2_ref, fw2_ref, fb2_ref, out_ref):
    dn = (((1,), (1,)), ((), ()))
    eps = 1e-5
    out = xo_ref[...] + _bdot(agg_ref[...], w2_ref[...], dn) + b2_ref[...]
    m1 = jnp.mean(out, axis=0, keepdims=True)
    v1 = jnp.mean((out - m1) ** 2, axis=0, keepdims=True)
    out = (out - m1) / jnp.sqrt(v1 + eps) * g1_ref[...] + bb1_ref[...]
    y = _bdot(out, fw1_ref[...], dn) + fb1_ref[...]
    y = jnp.maximum(y, 0.0)
    m2 = jnp.mean(y, axis=0, keepdims=True)
    v2 = jnp.mean((y - m2) ** 2, axis=0, keepdims=True)
    y = (y - m2) / jnp.sqrt(v2 + eps) * g2_ref[...] + bb2_ref[...]
    out_ref[...] = _bdot(y, fw2_ref[...], dn) + fb2_ref[...]


def kernel(x, batch, lin_s_W, lin_s_b, lin_h_W, lin_h_b, lin_out1_W,
           lin_out2_W, lin_out2_b, bn1_g, bn1_b, fc1_W, fc1_b,
           bn2_g, bn2_b, fc2_W, fc2_b):
    f32 = jnp.float32
    row = lambda v: v.reshape(1, -1).astype(f32)

    xp = jnp.pad(x, ((0, NPAD - N), (0, 0)))
    s_lp, table, xop = pl.pallas_call(
        _prep_body,
        out_shape=(
            jax.ShapeDtypeStruct((NPAD, S), f32),
            jax.ShapeDtypeStruct((NPAD, TW), f32),
            jax.ShapeDtypeStruct((NPAD, OUT), f32),
        ),
        interpret=_INTERPRET,
    )(xp, lin_s_W, row(lin_s_b), lin_h_W, row(lin_h_b), lin_out1_W)
    s_l = s_lp[:N]
    xo = xop[:N]

    st = s_l.T
    bq2 = batch.reshape(N, 1)
    bk2 = batch.reshape(1, N)

    nb = N // BQ
    full = lambda shape: pl.BlockSpec(shape, lambda i: (0, 0))
    nbr = pl.pallas_call(
        _knn_body,
        grid=(nb,),
        in_specs=[
            pl.BlockSpec((BQ, S), lambda i: (i, 0)),
            full((S, N)),
            pl.BlockSpec((BQ, 1), lambda i: (i, 0)),
            full((1, N)),
        ],
        out_specs=pl.BlockSpec((BQ, K), lambda i: (i, 0)),
        out_shape=jax.ShapeDtypeStruct((N, K), jnp.int32),
        interpret=_INTERPRET,
    )(s_l, st, bq2, bk2)

    nbr_flat = jnp.pad(nbr, ((0, NPAD - N), (0, 0))).reshape(-1)
    agg64 = _sc_agg(table, nbr_flat)[:N]

    w2p = jnp.zeros((OUT, AW), f32)
    w2p = w2p.at[:, 0:P].set(lin_out2_W[:, 0:P])
    w2p = w2p.at[:, 32:32 + P].set(lin_out2_W[:, P:2 * P])

    out = pl.pallas_call(
        _head_body,
        out_shape=jax.ShapeDtypeStruct((N, OUT), f32),
        interpret=_INTERPRET,
    )(xo, agg64, w2p, row(lin_out2_b), row(bn1_g), row(bn1_b),
      fc1_W, row(fc1_b), row(bn2_g), row(bn2_b), fc2_W, row(fc2_b))
    return out


# speedup vs baseline: 1.0622x; 1.0622x over previous
import functools

import jax
import jax.numpy as jnp
from jax import lax
from jax.experimental import pallas as pl
from jax.experimental.pallas import tpu as pltpu
from jax.experimental.pallas import tpu_sc as plsc

N = 10000
IN = 128
OUT = 32
S = 4
P = 22
K = 16
TW = 128
AW = 64
BQ = 200
BIG = 1e30
NPAD = 10240
NW = 32
BPW = NPAD // NW
SUB = 16
NSUB = BPW // SUB
_INTERPRET = False


def _bdot(a, b, dn):
    return jax.lax.dot_general(
        a.astype(jnp.bfloat16), b.astype(jnp.bfloat16), dn,
        preferred_element_type=jnp.float32)


def _prep_body(x_ref, ws_ref, bs_ref, wh_ref, bh_ref, w1_ref,
               s_ref, t_ref, xo_ref):
    x = x_ref[...]
    dn = (((1,), (1,)), ((), ()))
    s = _bdot(x, ws_ref[...], dn) + bs_ref[...]
    s_ref[...] = s
    h = _bdot(x, wh_ref[...], dn) + bh_ref[...]
    t_ref[:, 0:P] = h
    t_ref[:, P:P + S] = s
    t_ref[:, P + S:32] = jnp.zeros((x.shape[0], 32 - P - S), jnp.float32)
    t_ref[:, 32:TW] = jnp.zeros((x.shape[0], TW - 32), jnp.float32)
    xo_ref[...] = _bdot(x, w1_ref[...], dn)


def _knn_body(sq_ref, st_ref, bq_ref, bk_ref, nbr_ref):
    sq = sq_ref[...]
    st = st_ref[...]
    qn = jnp.sum(sq * sq, axis=1, keepdims=True)
    kn = jnp.sum(st * st, axis=0, keepdims=True)
    cross = _bdot(sq, st, (((1,), (0,)), ((), ())))
    d2 = qn + kn - 2.0 * cross
    mask = bq_ref[...] != bk_ref[...]
    d = jnp.where(mask, BIG, d2)
    iota = jax.lax.broadcasted_iota(jnp.int32, (BQ, N), 1)
    for k in range(K):
        idx = jnp.argmin(d, axis=1, keepdims=True)
        nbr_ref[:, k:k + 1] = idx.astype(jnp.int32)
        d = jnp.where(iota == idx, BIG, d)


def _take16(v, idx):
    return lax.gather(
        v, idx.reshape(16, 1),
        lax.GatherDimensionNumbers(offset_dims=(), collapsed_slice_dims=(0,),
                                   start_index_map=(0,)),
        (1,), mode=lax.GatherScatterMode.PROMISE_IN_BOUNDS)


def _sc_agg_body(t_hbm, nbr_hbm, out_hbm, idx_a, idx_b, rows_a, rows_b,
                 q_rows, out_v, sem_a, sem_b):
    wid = lax.axis_index("s") * 2 + lax.axis_index("c")
    base = wid * BPW
    idx_bufs = (idx_a, idx_b)
    row_bufs = (rows_a, rows_b)
    sems = (sem_a, sem_b)

    def fire(j, buf):
        pltpu.sync_copy(nbr_hbm.at[pl.ds((base + j * SUB) * K, SUB * K)],
                        idx_bufs[buf])
        return pltpu.async_copy(t_hbm.at[idx_bufs[buf]], row_bufs[buf],
                                sems[buf])

    lanes = lax.iota(jnp.int32, 16)
    rot1 = (lanes + 1) & 15
    rot2 = (lanes + 2) & 15
    rot3 = (lanes + 3) & 15

    fire(0, 0)
    fire(1, 1)

    def outer(j2, carry):
        for b in range(2):
            j = j2 * 2 + b
            rows_v = row_bufs[b]
            pltpu.make_async_copy(t_hbm.at[idx_bufs[b]], rows_v,
                                  sems[b]).wait()
            pltpu.sync_copy(t_hbm.at[pl.ds(base + j * SUB, SUB)], q_rows)

            def node(i, carry2, j=j, rows_v=rows_v):
                gi = j * SUB + i
                qv1 = q_rows[i, 16:32]
                acc0 = jnp.zeros((16,), jnp.float32)
                acc1 = jnp.zeros((16,), jnp.float32)
                mx0 = jnp.full((16,), -jnp.inf, jnp.float32)
                mx1 = jnp.full((16,), -jnp.inf, jnp.float32)
                for k in range(K):
                    h0 = rows_v[i * K + k, 0:16]
                    h1 = rows_v[i * K + k, 16:32]
                    diff = h1 - qv1
                    dsq = diff * diff
                    d2v = (dsq + _take16(dsq, rot1)
                           + _take16(dsq, rot2) + _take16(dsq, rot3))
                    w = jnp.exp(-10.0 * d2v)
                    wk = w[P - 16]
                    m0 = wk * h0
                    m1 = wk * h1
                    acc0 = acc0 + m0
                    acc1 = acc1 + m1
                    mx0 = jnp.maximum(mx0, m0)
                    mx1 = jnp.maximum(mx1, m1)
                out_v[gi, 0:16] = acc0 * (1.0 / K)
                out_v[gi, 16:32] = acc1 * (1.0 / K)
                out_v[gi, 32:48] = mx0
                out_v[gi, 48:64] = mx1
                return carry2

            lax.fori_loop(0, SUB, node, 0)

            @pl.when(j + 2 < NSUB)
            def _():
                fire(j + 2, b)
        return carry

    lax.fori_loop(0, NSUB // 2, outer, 0)
    pltpu.sync_copy(out_v, out_hbm.at[pl.ds(base, BPW)])


_sc_agg = functools.partial(
    pl.kernel,
    out_type=jax.ShapeDtypeStruct((NPAD, AW), jnp.float32),
    mesh=plsc.VectorSubcoreMesh(core_axis_name="c", subcore_axis_name="s"),
    scratch_types=[
        pltpu.VMEM((SUB * K,), jnp.int32),
        pltpu.VMEM((SUB * K,), jnp.int32),
        pltpu.VMEM((SUB * K, TW), jnp.float32),
        pltpu.VMEM((SUB * K, TW), jnp.float32),
        pltpu.VMEM((SUB, TW), jnp.float32),
        pltpu.VMEM((BPW, AW), jnp.float32),
        pltpu.SemaphoreType.DMA,
        pltpu.SemaphoreType.DMA,
    ],
)(_sc_agg_body)


def _head_body(xo_ref, agg_ref, w2_ref, b2_ref, g1_ref, bb1_ref,
               fw1_ref, fb1_ref, g2_ref, bb2_ref, fw2_ref, fb2_ref, out_ref):
    dn = (((1,), (1,)), ((), ()))
    eps = 1e-5
    out = xo_ref[...] + _bdot(agg_ref[...], w2_ref[...], dn) + b2_ref[...]
    m1 = jnp.mean(out, axis=0, keepdims=True)
    v1 = jnp.mean((out - m1) ** 2, axis=0, keepdims=True)
    out = (out - m1) / jnp.sqrt(v1 + eps) * g1_ref[...] + bb1_ref[...]
    y = _bdot(out, fw1_ref[...], dn) + fb1_ref[...]
    y = jnp.maximum(y, 0.0)
    m2 = jnp.mean(y, axis=0, keepdims=True)
    v2 = jnp.mean((y - m2) ** 2, axis=0, keepdims=True)
    y = (y - m2) / jnp.sqrt(v2 + eps) * g2_ref[...] + bb2_ref[...]
    out_ref[...] = _bdot(y, fw2_ref[...], dn) + fb2_ref[...]


def kernel(x, batch, lin_s_W, lin_s_b, lin_h_W, lin_h_b, lin_out1_W,
           lin_out2_W, lin_out2_b, bn1_g, bn1_b, fc1_W, fc1_b,
           bn2_g, bn2_b, fc2_W, fc2_b):
    f32 = jnp.float32
    row = lambda v: v.reshape(1, -1).astype(f32)

    xp = jnp.pad(x, ((0, NPAD - N), (0, 0)))
    s_lp, table, xop = pl.pallas_call(
        _prep_body,
        out_shape=(
            jax.ShapeDtypeStruct((NPAD, S), f32),
            jax.ShapeDtypeStruct((NPAD, TW), f32),
            jax.ShapeDtypeStruct((NPAD, OUT), f32),
        ),
        interpret=_INTERPRET,
    )(xp, lin_s_W, row(lin_s_b), lin_h_W, row(lin_h_b), lin_out1_W)
    s_l = s_lp[:N]
    xo = xop[:N]

    st = s_l.T
    bq2 = batch.reshape(N, 1)
    bk2 = batch.reshape(1, N)

    nb = N // BQ
    full = lambda shape: pl.BlockSpec(shape, lambda i: (0, 0))
    nbr = pl.pallas_call(
        _knn_body,
        grid=(nb,),
        in_specs=[
            pl.BlockSpec((BQ, S), lambda i: (i, 0)),
            full((S, N)),
            pl.BlockSpec((BQ, 1), lambda i: (i, 0)),
            full((1, N)),
        ],
        out_specs=pl.BlockSpec((BQ, K), lambda i: (i, 0)),
        out_shape=jax.ShapeDtypeStruct((N, K), jnp.int32),
        interpret=_INTERPRET,
    )(s_l, st, bq2, bk2)

    nbr_flat = jnp.pad(nbr, ((0, NPAD - N), (0, 0))).reshape(-1)
    agg64 = _sc_agg(table, nbr_flat)[:N]

    w2p = jnp.zeros((OUT, AW), f32)
    w2p = w2p.at[:, 0:P].set(lin_out2_W[:, 0:P])
    w2p = w2p.at[:, 32:32 + P].set(lin_out2_W[:, P:2 * P])

    out = pl.pallas_call(
        _head_body,
        out_shape=jax.ShapeDtypeStruct((N, OUT), f32),
        interpret=_INTERPRET,
    )(xo, agg64, w2p, row(lin_out2_b), row(bn1_g), row(bn1_b),
      fc1_W, row(fc1_b), row(bn2_g), row(bn2_b), fc2_W, row(fc2_b))
    return out
